# baseline (speedup 1.0000x reference)
import jax
import jax.numpy as jnp
from jax import lax
from jax.experimental import pallas as pl
from jax.experimental.pallas import tpu as pltpu
from jax.experimental.pallas import tpu_sc as plsc

D = 1024
L = 16
NW = 32
ROWS = 8192
ROWS_PER_W = ROWS // NW
R = 16
N_TILES = ROWS_PER_W // R
N_CHUNKS = D // L
NBUF = 2


def _permute_body(x_hbm, perm_hbm, z_hbm, perm_v, *bufs):
    xins = bufs[0:NBUF]
    outs = bufs[NBUF:2 * NBUF]
    in_sems = bufs[2 * NBUF:3 * NBUF]
    out_sems = bufs[3 * NBUF:4 * NBUF]
    wid = lax.axis_index("s") * 2 + lax.axis_index("c")
    pltpu.sync_copy(perm_hbm, perm_v)
    base = wid * ROWS_PER_W

    for b in range(NBUF):
        pltpu.async_copy(x_hbm.at[pl.ds(base + b * R, R)],
                         xins[b], in_sems[b])

    @pl.loop(0, N_TILES, step=NBUF)
    def _t(t):
        for b in range(NBUF):
            tile = t + b
            row0 = base + tile * R
            xin_b = xins[b]
            out_b = outs[b]
            pltpu.make_async_copy(x_hbm.at[pl.ds(row0, R)],
                                  xin_b, in_sems[b]).wait()

            @pl.when(tile >= NBUF)
            def _():
                pltpu.make_async_copy(out_b, z_hbm.at[pl.ds(row0, R)],
                                      out_sems[b]).wait()

            @plsc.parallel_loop(0, N_CHUNKS, unroll=4)
            def _chunk(c):
                idx = perm_v[pl.ds(c * L, L)]
                for r in range(R):
                    row = jnp.full((L,), r, dtype=jnp.int32)
                    vals = plsc.load_gather(xin_b, [row, idx])
                    out_b[r, pl.ds(c * L, L)] = vals

            pltpu.async_copy(out_b, z_hbm.at[pl.ds(row0, R)], out_sems[b])

            nxt = tile + NBUF
            @pl.when(nxt < N_TILES)
            def _():
                pltpu.async_copy(x_hbm.at[pl.ds(base + nxt * R, R)],
                                 xin_b, in_sems[b])

    for b in range(NBUF):
        pltpu.make_async_copy(outs[b], z_hbm.at[pl.ds(base, R)],
                              out_sems[b]).wait()


@jax.jit
def _permute(x2, perm):
    mesh = plsc.VectorSubcoreMesh(core_axis_name="c", subcore_axis_name="s")
    return pl.kernel(
        _permute_body,
        out_type=jax.ShapeDtypeStruct((ROWS, D), jnp.float32),
        mesh=mesh,
        compiler_params=pltpu.CompilerParams(needs_layout_passes=False),
        scratch_types=(
            [pltpu.VMEM((D,), jnp.int32)]
            + [pltpu.VMEM((R, D), jnp.float32)] * (2 * NBUF)
            + [pltpu.SemaphoreType.DMA] * (2 * NBUF)
        ),
    )(x2, perm)


def kernel(x, perm):
    x2 = x.reshape(ROWS, D)
    z2 = _permute(x2, perm.astype(jnp.int32))
    z = z2.reshape(x.shape)
    log_det = jnp.zeros(x.shape[:-1], dtype=x.dtype)
    return (z, log_det)


# speedup vs baseline: 1.0343x; 1.0343x over previous
import jax
import jax.numpy as jnp
from jax import lax
from jax.experimental import pallas as pl
from jax.experimental.pallas import tpu as pltpu
from jax.experimental.pallas import tpu_sc as plsc

D = 1024
L = 16
NW = 32
ROWS = 8192
ROWS_PER_W = ROWS // NW
R = 4
N_TILES = ROWS_PER_W // R
N_CHUNKS = D // L
NBUF = 8


def _permute_body(x_hbm, perm_hbm, z_hbm, perm_v, *bufs):
    xins = bufs[0:NBUF]
    outs = bufs[NBUF:2 * NBUF]
    in_sems = bufs[2 * NBUF:3 * NBUF]
    out_sems = bufs[3 * NBUF:4 * NBUF]
    wid = lax.axis_index("s") * 2 + lax.axis_index("c")
    pltpu.sync_copy(perm_hbm, perm_v)
    base = wid * ROWS_PER_W

    for b in range(NBUF):
        pltpu.async_copy(x_hbm.at[pl.ds(base + b * R, R)],
                         xins[b], in_sems[b])

    @pl.loop(0, N_TILES, step=NBUF)
    def _t(t):
        for b in range(NBUF):
            tile = t + b
            row0 = base + tile * R
            xin_b = xins[b]
            out_b = outs[b]
            pltpu.make_async_copy(x_hbm.at[pl.ds(row0, R)],
                                  xin_b, in_sems[b]).wait()

            @pl.when(tile >= NBUF)
            def _():
                pltpu.make_async_copy(out_b, z_hbm.at[pl.ds(row0, R)],
                                      out_sems[b]).wait()

            @plsc.parallel_loop(0, N_CHUNKS, unroll=4)
            def _chunk(c):
                idx = perm_v[pl.ds(c * L, L)]
                for r in range(R):
                    row = jnp.full((L,), r, dtype=jnp.int32)
                    vals = plsc.load_gather(xin_b, [row, idx])
                    out_b[r, pl.ds(c * L, L)] = vals

            pltpu.async_copy(out_b, z_hbm.at[pl.ds(row0, R)], out_sems[b])

            nxt = tile + NBUF
            @pl.when(nxt < N_TILES)
            def _():
                pltpu.async_copy(x_hbm.at[pl.ds(base + nxt * R, R)],
                                 xin_b, in_sems[b])

    for b in range(NBUF):
        pltpu.make_async_copy(outs[b], z_hbm.at[pl.ds(base, R)],
                              out_sems[b]).wait()


@jax.jit
def _permute(x2, perm):
    mesh = plsc.VectorSubcoreMesh(core_axis_name="c", subcore_axis_name="s")
    return pl.kernel(
        _permute_body,
        out_type=jax.ShapeDtypeStruct((ROWS, D), jnp.float32),
        mesh=mesh,
        compiler_params=pltpu.CompilerParams(needs_layout_passes=False),
        scratch_types=(
            [pltpu.VMEM((D,), jnp.int32)]
            + [pltpu.VMEM((R, D), jnp.float32)] * (2 * NBUF)
            + [pltpu.SemaphoreType.DMA] * (2 * NBUF)
        ),
    )(x2, perm)


def kernel(x, perm):
    x2 = x.reshape(ROWS, D)
    z2 = _permute(x2, perm.astype(jnp.int32))
    z = z2.reshape(x.shape)
    log_det = jnp.zeros(x.shape[:-1], dtype=x.dtype)
    return (z, log_det)


# speedup vs baseline: 1.0514x; 1.0165x over previous
import jax
import jax.numpy as jnp
from jax import lax
from jax.experimental import pallas as pl
from jax.experimental.pallas import tpu as pltpu
from jax.experimental.pallas import tpu_sc as plsc

D = 1024
L = 16
NW = 32
ROWS = 8192
ROWS_PER_W = ROWS // NW
R = 8
N_TILES = ROWS_PER_W // R
N_CHUNKS = D // L
NBUF = 4


def _permute_body(x_hbm, perm_hbm, z_hbm, perm_v, *bufs):
    xins = bufs[0:NBUF]
    outs = bufs[NBUF:2 * NBUF]
    in_sems = bufs[2 * NBUF:3 * NBUF]
    out_sems = bufs[3 * NBUF:4 * NBUF]
    wid = lax.axis_index("s") * 2 + lax.axis_index("c")
    base = wid * ROWS_PER_W

    for b in range(NBUF):
        pltpu.async_copy(x_hbm.at[pl.ds(base + b * R, R)],
                         xins[b], in_sems[b])
    pltpu.sync_copy(perm_hbm, perm_v)

    @pl.loop(0, N_TILES, step=NBUF)
    def _t(t):
        for b in range(NBUF):
            tile = t + b
            row0 = base + tile * R
            xin_b = xins[b]
            out_b = outs[b]
            pltpu.make_async_copy(x_hbm.at[pl.ds(row0, R)],
                                  xin_b, in_sems[b]).wait()

            @pl.when(tile >= NBUF)
            def _():
                pltpu.make_async_copy(out_b, z_hbm.at[pl.ds(row0, R)],
                                      out_sems[b]).wait()

            @plsc.parallel_loop(0, N_CHUNKS, unroll=4)
            def _chunk(c):
                idx = perm_v[pl.ds(c * L, L)]
                for r in range(R):
                    row = jnp.full((L,), r, dtype=jnp.int32)
                    vals = plsc.load_gather(xin_b, [row, idx])
                    out_b[r, pl.ds(c * L, L)] = vals

            pltpu.async_copy(out_b, z_hbm.at[pl.ds(row0, R)], out_sems[b])

            nxt = tile + NBUF
            @pl.when(nxt < N_TILES)
            def _():
                pltpu.async_copy(x_hbm.at[pl.ds(base + nxt * R, R)],
                                 xin_b, in_sems[b])

    for b in range(NBUF):
        pltpu.make_async_copy(outs[b], z_hbm.at[pl.ds(base, R)],
                              out_sems[b]).wait()


@jax.jit
def _permute(x2, perm):
    mesh = plsc.VectorSubcoreMesh(core_axis_name="c", subcore_axis_name="s")
    return pl.kernel(
        _permute_body,
        out_type=jax.ShapeDtypeStruct((ROWS, D), jnp.float32),
        mesh=mesh,
        compiler_params=pltpu.CompilerParams(needs_layout_passes=False),
        scratch_types=(
            [pltpu.VMEM((D,), jnp.int32)]
            + [pltpu.VMEM((R, D), jnp.float32)] * (2 * NBUF)
            + [pltpu.SemaphoreType.DMA] * (2 * NBUF)
        ),
    )(x2, perm)


def kernel(x, perm):
    x2 = x.reshape(ROWS, D)
    z2 = _permute(x2, perm.astype(jnp.int32))
    z = z2.reshape(x.shape)
    log_det = jnp.zeros(x.shape[:-1], dtype=x.dtype)
    return (z, log_det)


# speedup vs baseline: 1.0549x; 1.0033x over previous
import jax
import jax.numpy as jnp
from jax import lax
from jax.experimental import pallas as pl
from jax.experimental.pallas import tpu as pltpu
from jax.experimental.pallas import tpu_sc as plsc

D = 1024
L = 16
NW = 32
ROWS = 8192
ROWS_PER_W = ROWS // NW
R = 8
N_TILES = ROWS_PER_W // R
N_CHUNKS = D // L
NBUF = 4


def _permute_body(x_hbm, perm_hbm, z_hbm, perm_v, *bufs):
    xins = bufs[0:NBUF]
    outs = bufs[NBUF:2 * NBUF]
    in_sems = bufs[2 * NBUF:3 * NBUF]
    out_sems = bufs[3 * NBUF:4 * NBUF]
    wid = lax.axis_index("s") * 2 + lax.axis_index("c")

    for b in range(NBUF):
        pltpu.async_copy(x_hbm.at[pl.ds((b * NW + wid) * R, R)],
                         xins[b], in_sems[b])
    pltpu.sync_copy(perm_hbm, perm_v)

    @pl.loop(0, N_TILES, step=NBUF)
    def _t(t):
        for b in range(NBUF):
            tile = t + b
            row0 = (tile * NW + wid) * R
            xin_b = xins[b]
            out_b = outs[b]
            pltpu.make_async_copy(x_hbm.at[pl.ds(row0, R)],
                                  xin_b, in_sems[b]).wait()

            @pl.when(tile >= NBUF)
            def _():
                pltpu.make_async_copy(out_b, z_hbm.at[pl.ds(row0, R)],
                                      out_sems[b]).wait()

            @plsc.parallel_loop(0, N_CHUNKS, unroll=4)
            def _chunk(c):
                idx = perm_v[pl.ds(c * L, L)]
                for r in range(R):
                    row = jnp.full((L,), r, dtype=jnp.int32)
                    vals = plsc.load_gather(xin_b, [row, idx])
                    out_b[r, pl.ds(c * L, L)] = vals

            pltpu.async_copy(out_b, z_hbm.at[pl.ds(row0, R)], out_sems[b])

            nxt = tile + NBUF
            @pl.when(nxt < N_TILES)
            def _():
                pltpu.async_copy(x_hbm.at[pl.ds((nxt * NW + wid) * R, R)],
                                 xin_b, in_sems[b])

    for b in range(NBUF):
        pltpu.make_async_copy(outs[b], z_hbm.at[pl.ds(wid * R, R)],
                              out_sems[b]).wait()


@jax.jit
def _permute(x2, perm):
    mesh = plsc.VectorSubcoreMesh(core_axis_name="c", subcore_axis_name="s")
    return pl.kernel(
        _permute_body,
        out_type=jax.ShapeDtypeStruct((ROWS, D), jnp.float32),
        mesh=mesh,
        compiler_params=pltpu.CompilerParams(needs_layout_passes=False),
        scratch_types=(
            [pltpu.VMEM((D,), jnp.int32)]
            + [pltpu.VMEM((R, D), jnp.float32)] * (2 * NBUF)
            + [pltpu.SemaphoreType.DMA] * (2 * NBUF)
        ),
    )(x2, perm)


def kernel(x, perm):
    x2 = x.reshape(ROWS, D)
    z2 = _permute(x2, perm.astype(jnp.int32))
    z = z2.reshape(x.shape)
    log_det = jnp.zeros(x.shape[:-1], dtype=x.dtype)
    return (z, log_det)
